# Initial kernel scaffold; baseline (speedup 1.0000x reference)
import jax
import jax.numpy as jnp
from jax.experimental import pallas as pl


def kernel(hidden_states, W):
    raise NotImplementedError("write your pallas kernel here")


# speedup vs baseline: 2.4605x; 2.4605x over previous
import jax
import jax.numpy as jnp
from jax.experimental import pallas as pl
from jax.experimental.pallas import tpu as pltpu

NUM_EXPERTS = 64
CAPACITY = 80
BLOCK_S = 512


def _router_body(x_ref, w_ref, oh_ref, pmax_ref, logits_ref, counts_ref):
    sb = pl.program_id(1)

    @pl.when(sb == 0)
    def _():
        counts_ref[...] = jnp.zeros_like(counts_ref)

    x = x_ref[0]
    w = w_ref[...]
    logits = jax.lax.dot_general(
        x, w, (((1,), (1,)), ((), ())), preferred_element_type=jnp.float32
    )
    logits_ref[0] = logits

    m = jnp.max(logits, axis=-1, keepdims=True)
    ssum = jnp.sum(jnp.exp(logits - m), axis=-1, keepdims=True)
    pmax_ref[0] = 1.0 / ssum

    idx = jnp.argmax(logits, axis=-1)
    iota = jax.lax.broadcasted_iota(jnp.int32, (BLOCK_S, NUM_EXPERTS), 1)
    oh = (iota == idx[:, None]).astype(jnp.int32)

    r = jax.lax.broadcasted_iota(jnp.int32, (BLOCK_S, BLOCK_S), 0)
    c = jax.lax.broadcasted_iota(jnp.int32, (BLOCK_S, BLOCK_S), 1)
    tril = (r >= c).astype(jnp.float32)
    oh_f = oh.astype(jnp.float32)
    csum = jax.lax.dot_general(
        tril, oh_f, (((1,), (0,)), ((), ())), preferred_element_type=jnp.float32
    ).astype(jnp.int32)
    priority = csum + counts_ref[...]
    keep = (priority <= CAPACITY).astype(jnp.int32)
    oh_ref[0] = oh * keep
    counts_ref[...] = counts_ref[...] + jnp.sum(oh, axis=0, keepdims=True)


def kernel(hidden_states, W):
    B, S, H = hidden_states.shape
    E = W.shape[0]
    grid = (B, S // BLOCK_S)

    out_shapes = (
        jax.ShapeDtypeStruct((B, S, E), jnp.int32),
        jax.ShapeDtypeStruct((B, S, 1), jnp.float32),
        jax.ShapeDtypeStruct((B, S, E), jnp.float32),
    )
    oh, pmax, logits = pl.pallas_call(
        _router_body,
        grid=grid,
        in_specs=[
            pl.BlockSpec((1, BLOCK_S, H), lambda b, s: (b, s, 0)),
            pl.BlockSpec((E, H), lambda b, s: (0, 0)),
        ],
        out_specs=[
            pl.BlockSpec((1, BLOCK_S, E), lambda b, s: (b, s, 0)),
            pl.BlockSpec((1, BLOCK_S, 1), lambda b, s: (b, s, 0)),
            pl.BlockSpec((1, BLOCK_S, E), lambda b, s: (b, s, 0)),
        ],
        out_shape=out_shapes,
        scratch_shapes=[pltpu.VMEM((1, NUM_EXPERTS), jnp.int32)],
        compiler_params=pltpu.CompilerParams(
            dimension_semantics=("arbitrary", "arbitrary"),
        ),
    )(hidden_states, W)
    return (oh, pmax, logits)


# speedup vs baseline: 2.6493x; 1.0767x over previous
import jax
import jax.numpy as jnp
from jax.experimental import pallas as pl
from jax.experimental.pallas import tpu as pltpu

NUM_EXPERTS = 64
CAPACITY = 80
BLOCK_S = 1024


def _router_body(x_ref, w_ref, oh_ref, pmax_ref, logits_ref, counts_ref):
    sb = pl.program_id(1)

    @pl.when(sb == 0)
    def _():
        counts_ref[...] = jnp.zeros_like(counts_ref)

    x = x_ref[0]
    w = w_ref[...]
    logits = jax.lax.dot_general(
        x, w, (((1,), (1,)), ((), ())), preferred_element_type=jnp.float32
    )
    logits_ref[0] = logits

    m = jnp.max(logits, axis=-1, keepdims=True)
    ssum = jnp.sum(jnp.exp(logits - m), axis=-1, keepdims=True)
    pmax_ref[0] = 1.0 / ssum

    idx = jnp.argmax(logits, axis=-1)
    iota = jax.lax.broadcasted_iota(jnp.int32, (BLOCK_S, NUM_EXPERTS), 1)
    oh = (iota == idx[:, None]).astype(jnp.int32)

    r = jax.lax.broadcasted_iota(jnp.int32, (BLOCK_S, BLOCK_S), 0)
    c = jax.lax.broadcasted_iota(jnp.int32, (BLOCK_S, BLOCK_S), 1)
    tril = (r >= c).astype(jnp.float32)
    oh_f = oh.astype(jnp.float32)
    csum = jax.lax.dot_general(
        tril, oh_f, (((1,), (0,)), ((), ())), preferred_element_type=jnp.float32
    ).astype(jnp.int32)
    priority = csum + counts_ref[...]
    keep = (priority <= CAPACITY).astype(jnp.int32)
    oh_ref[0] = oh * keep
    counts_ref[...] = counts_ref[...] + jnp.sum(oh, axis=0, keepdims=True)


def kernel(hidden_states, W):
    B, S, H = hidden_states.shape
    E = W.shape[0]
    grid = (B, S // BLOCK_S)

    out_shapes = (
        jax.ShapeDtypeStruct((B, S, E), jnp.int32),
        jax.ShapeDtypeStruct((B, S, 1), jnp.float32),
        jax.ShapeDtypeStruct((B, S, E), jnp.float32),
    )
    oh, pmax, logits = pl.pallas_call(
        _router_body,
        grid=grid,
        in_specs=[
            pl.BlockSpec((1, BLOCK_S, H), lambda b, s: (b, s, 0)),
            pl.BlockSpec((E, H), lambda b, s: (0, 0)),
        ],
        out_specs=[
            pl.BlockSpec((1, BLOCK_S, E), lambda b, s: (b, s, 0)),
            pl.BlockSpec((1, BLOCK_S, 1), lambda b, s: (b, s, 0)),
            pl.BlockSpec((1, BLOCK_S, E), lambda b, s: (b, s, 0)),
        ],
        out_shape=out_shapes,
        scratch_shapes=[pltpu.VMEM((1, NUM_EXPERTS), jnp.int32)],
        compiler_params=pltpu.CompilerParams(
            dimension_semantics=("arbitrary", "arbitrary"),
        ),
    )(hidden_states, W)
    return (oh, pmax, logits)


# speedup vs baseline: 2.8965x; 1.0933x over previous
import jax
import jax.numpy as jnp
from jax.experimental import pallas as pl
from jax.experimental.pallas import tpu as pltpu

NUM_EXPERTS = 64
CAPACITY = 80
BLOCK_S = 2048
CHUNK = 512


def _router_body(x_ref, w_ref, oh_ref, pmax_ref, logits_ref, counts_ref):
    sb = pl.program_id(1)

    @pl.when(sb == 0)
    def _():
        counts_ref[...] = jnp.zeros_like(counts_ref)

    x = x_ref[0]
    w = w_ref[...]
    logits = jax.lax.dot_general(
        x, w, (((1,), (1,)), ((), ())), preferred_element_type=jnp.float32
    )
    logits_ref[0] = logits

    m = jnp.max(logits, axis=-1, keepdims=True)
    ssum = jnp.sum(jnp.exp(logits - m), axis=-1, keepdims=True)
    pmax_ref[0] = 1.0 / ssum

    idx = jnp.argmax(logits, axis=-1)
    iota = jax.lax.broadcasted_iota(jnp.int32, (BLOCK_S, NUM_EXPERTS), 1)
    oh = (iota == idx[:, None]).astype(jnp.int32)

    r = jax.lax.broadcasted_iota(jnp.int32, (CHUNK, CHUNK), 0)
    c = jax.lax.broadcasted_iota(jnp.int32, (CHUNK, CHUNK), 1)
    tril = (r >= c).astype(jnp.float32)

    counts = counts_ref[...]
    for ci in range(BLOCK_S // CHUNK):
        ohc = oh[ci * CHUNK:(ci + 1) * CHUNK]
        csum = jax.lax.dot_general(
            tril, ohc.astype(jnp.float32), (((1,), (0,)), ((), ())),
            preferred_element_type=jnp.float32,
        ).astype(jnp.int32)
        priority = csum + counts
        keep = (priority <= CAPACITY).astype(jnp.int32)
        oh_ref[0, ci * CHUNK:(ci + 1) * CHUNK, :] = ohc * keep
        counts = counts + csum[CHUNK - 1:CHUNK, :]
    counts_ref[...] = counts


def kernel(hidden_states, W):
    B, S, H = hidden_states.shape
    E = W.shape[0]
    grid = (B, S // BLOCK_S)

    out_shapes = (
        jax.ShapeDtypeStruct((B, S, E), jnp.int32),
        jax.ShapeDtypeStruct((B, S, 1), jnp.float32),
        jax.ShapeDtypeStruct((B, S, E), jnp.float32),
    )
    oh, pmax, logits = pl.pallas_call(
        _router_body,
        grid=grid,
        in_specs=[
            pl.BlockSpec((1, BLOCK_S, H), lambda b, s: (b, s, 0)),
            pl.BlockSpec((E, H), lambda b, s: (0, 0)),
        ],
        out_specs=[
            pl.BlockSpec((1, BLOCK_S, E), lambda b, s: (b, s, 0)),
            pl.BlockSpec((1, BLOCK_S, 1), lambda b, s: (b, s, 0)),
            pl.BlockSpec((1, BLOCK_S, E), lambda b, s: (b, s, 0)),
        ],
        out_shape=out_shapes,
        scratch_shapes=[pltpu.VMEM((1, NUM_EXPERTS), jnp.int32)],
        compiler_params=pltpu.CompilerParams(
            dimension_semantics=("arbitrary", "arbitrary"),
        ),
    )(hidden_states, W)
    return (oh, pmax, logits)
